# baseline (speedup 1.0000x reference)
import jax
import jax.numpy as jnp
from jax.experimental import pallas as pl
from jax.experimental.pallas import tpu as pltpu

_EMBED = 1024
_HEADS = 16
_HDIM = 64
_SEQ = 2048
_QCHUNK = 512
_PAIRS = _HEADS // 2
_PCHUNK = 256
_PSTEPS = _SEQ // _PCHUNK
_EXP2_SCALE = 1.4426950408889634 / 8.0
_EXP2_CLAMP = 120.0


def _mha_body(xq_ref, xk_ref, xv_ref, w3_ref, b3_ref, wo_ref, bo_ref,
              out_ref, qkv_ref, acc_ref):
    j = pl.program_id(0)
    f32 = jnp.float32
    bf16 = jnp.bfloat16
    dn = (((1,), (1,)), ((), ()))

    @pl.when(j < _PSTEPS)
    def _proj():
        row0 = j * _PCHUNK
        for t, x_ref in enumerate((xq_ref, xk_ref, xv_ref)):
            xb = x_ref[...].astype(bf16)
            w_t = w3_ref[t * _EMBED:(t + 1) * _EMBED]
            p = jax.lax.dot_general(xb, w_t, dn, preferred_element_type=f32)
            p = p + b3_ref[t:t + 1]
            qkv_ref[pl.ds(row0, _PCHUNK), t * _EMBED:(t + 1) * _EMBED] = (
                p.astype(bf16))

    @pl.when(jnp.logical_and(j >= _PSTEPS, j < _PSTEPS + _PAIRS))
    def _heads():
        ones_col = (jax.lax.broadcasted_iota(jnp.int32, (_SEQ, _HDIM), 1) == 0)
        lane0 = (j - _PSTEPS) * 2 * _HDIM
        q_pair = qkv_ref[:, pl.ds(lane0, 2 * _HDIM)]
        k_pair = qkv_ref[:, pl.ds(_EMBED + lane0, 2 * _HDIM)]
        v_pair = qkv_ref[:, pl.ds(2 * _EMBED + lane0, 2 * _HDIM)]
        for hh in range(2):
            sl_h = slice(hh * _HDIM, (hh + 1) * _HDIM)
            q_h = q_pair[:, sl_h]
            k_h = k_pair[:, sl_h]
            v_ext = jnp.concatenate([v_pair[:, sl_h], ones_col.astype(bf16)],
                                    axis=1)
            for i in range(_SEQ // _QCHUNK):
                qc = q_h[i * _QCHUNK:(i + 1) * _QCHUNK]
                s = jax.lax.dot_general(qc, k_h, dn, preferred_element_type=f32)
                e = jnp.exp2(jnp.minimum(s * _EXP2_SCALE, _EXP2_CLAMP)).astype(bf16)
                o_ext = jnp.dot(e, v_ext, preferred_element_type=f32)
                r = o_ext[:, _HDIM:_HDIM + 1]
                acc_ref[pl.ds(i * _QCHUNK, _QCHUNK),
                        pl.ds(2 * _HDIM * (2 * (j - _PSTEPS) + hh), 2 * _HDIM)] = (
                    (o_ext / r).astype(bf16))

    @pl.when(j == _PSTEPS + _PAIRS)
    def _outproj():
        out_ref[...] = jnp.dot(acc_ref[...], wo_ref[...],
                               preferred_element_type=f32) + bo_ref[...]


def kernel(query, key, value, in_proj_weight, in_proj_bias,
           ln_q_g, ln_q_b, gp_q_w, gp_q_b,
           ln_k_g, ln_k_b, gp_k_w, gp_k_b,
           ln_v_g, ln_v_b, gp_v_w, gp_v_b,
           out_w, out_b):
    del ln_q_g, ln_q_b, gp_q_w, gp_q_b, ln_k_g, ln_k_b, gp_k_w, gp_k_b
    del ln_v_g, ln_v_b, gp_v_w, gp_v_b
    bf16 = jnp.bfloat16
    xq = query[:, 0, :]
    xk = key[:, 0, :]
    xv = value[:, 0, :]
    w3 = in_proj_weight.astype(bf16)
    b3 = in_proj_bias.reshape(3, _EMBED)
    bo = out_b.reshape(1, _EMBED)
    wo_big = jnp.pad(out_w.T.astype(bf16).reshape(_HEADS, _HDIM, _EMBED),
                     ((0, 0), (0, _HDIM), (0, 0))).reshape(2 * _EMBED, _EMBED)

    def xmap(j):
        return (jnp.minimum(j, _PSTEPS - 1), 0)

    out2d = pl.pallas_call(
        _mha_body,
        grid=(_PSTEPS + _PAIRS + 1,),
        in_specs=[
            pl.BlockSpec((_PCHUNK, _EMBED), xmap),
            pl.BlockSpec((_PCHUNK, _EMBED), xmap),
            pl.BlockSpec((_PCHUNK, _EMBED), xmap),
            pl.BlockSpec((3 * _EMBED, _EMBED), lambda j: (0, 0)),
            pl.BlockSpec((3, _EMBED), lambda j: (0, 0)),
            pl.BlockSpec((2 * _EMBED, _EMBED), lambda j: (0, 0)),
            pl.BlockSpec((1, _EMBED), lambda j: (0, 0)),
        ],
        out_specs=pl.BlockSpec((_SEQ, _EMBED), lambda j: (0, 0)),
        out_shape=jax.ShapeDtypeStruct((_SEQ, _EMBED), jnp.float32),
        scratch_shapes=[pltpu.VMEM((_SEQ, 3 * _EMBED), bf16),
                        pltpu.VMEM((_SEQ, 2 * _EMBED), bf16)],
    )(xq, xk, xv, w3, b3, wo_big, bo)
    return out2d[:, None, :]
